# baseline (speedup 1.0000x reference)
import functools

import jax
import jax.numpy as jnp
from jax import lax
from jax.experimental import pallas as pl
from jax.experimental.pallas import tpu as pltpu
from jax.experimental.pallas import tpu_sc as plsc

SEM_DIM = 64
ENT_DIM = 64

_NC = 2
_NS = 16
_NW = _NC * _NS

_RC = 64
_GI = 128


def _sc_gather(center_emb, context_emb, center_ids, pos_ids, neg_ids_flat, K):
    B = center_ids.shape[0]
    DC = center_emb.shape[1]
    D = context_emb.shape[1]
    b_w = B // _NW
    n_ch = b_w // _RC
    ng = _RC * K // _GI

    mesh = plsc.VectorSubcoreMesh(
        core_axis_name="c", subcore_axis_name="s",
        num_cores=_NC, num_subcores=_NS)

    @functools.partial(
        pl.kernel,
        out_type=(
            jax.ShapeDtypeStruct((B, DC), jnp.float32),
            jax.ShapeDtypeStruct((B, D), jnp.float32),
            jax.ShapeDtypeStruct((B * K, D), jnp.float32),
        ),
        mesh=mesh,
        scratch_types=[
            pltpu.VMEM((_RC,), jnp.int32),
            pltpu.VMEM((_RC,), jnp.int32),
            pltpu.VMEM((_RC * K,), jnp.int32),
            pltpu.VMEM((_RC, DC), jnp.float32),
            pltpu.VMEM((_RC, D), jnp.float32),
            pltpu.VMEM((_RC * K, D), jnp.float32),
            pltpu.SemaphoreType.DMA,
            pltpu.SemaphoreType.DMA,
        ],
        compiler_params=pltpu.CompilerParams(use_tc_tiling_on_sc=False),
    )
    def k(cemb, xemb, cids, pids, nids, cout, pout, nout,
          cidx, pidx, nidx, crows, prows, nrows, sem_g, sem_w):
        wid = lax.axis_index("s") * _NC + lax.axis_index("c")
        for ch in range(n_ch):
            base = wid * b_w + ch * _RC
            nbase = base * K
            pltpu.sync_copy(cids.at[pl.ds(base, _RC)], cidx)
            pltpu.sync_copy(pids.at[pl.ds(base, _RC)], pidx)
            pltpu.sync_copy(nids.at[pl.ds(nbase, _RC * K)], nidx)
            copies = [
                pltpu.async_copy(cemb.at[cidx], crows, sem_g),
                pltpu.async_copy(xemb.at[pidx], prows, sem_g),
            ]
            for g in range(ng):
                copies.append(pltpu.async_copy(
                    xemb.at[nidx.at[pl.ds(g * _GI, _GI)]],
                    nrows.at[pl.ds(g * _GI, _GI)], sem_g))
            for c in copies:
                c.wait()
            writes = [
                pltpu.async_copy(crows, cout.at[pl.ds(base, _RC)], sem_w),
                pltpu.async_copy(prows, pout.at[pl.ds(base, _RC)], sem_w),
                pltpu.async_copy(nrows, nout.at[pl.ds(nbase, _RC * K)], sem_w),
            ]
            for c in writes:
                c.wait()

    return k(center_emb, context_emb, center_ids, pos_ids, neg_ids_flat)


def _tc_body(cref, pref, nref, tref, wtref, bref, oref):
    i = pl.program_id(0)
    c = cref[...]
    sem = c[:, :SEM_DIM]
    ent = c[:, SEM_DIM:]
    pos = pref[...]
    negf = nref[...]
    r = sem.shape[0]
    nega = negf[:, :SEM_DIM].reshape(r, -1, SEM_DIM)
    negb = negf[:, SEM_DIM:].reshape(r, -1, SEM_DIM)
    ps = jnp.sum(sem * pos, axis=1)
    nsa = jnp.sum(nega * sem[:, None, :], axis=2)
    nsb = jnp.sum(negb * sem[:, None, :], axis=2)

    def log_sigmoid(x):
        return jnp.minimum(x, 0.0) - jnp.log(1.0 + jnp.exp(-jnp.abs(x)))

    s_skip = (jnp.sum(log_sigmoid(ps)) + jnp.sum(log_sigmoid(-nsa))
              + jnp.sum(log_sigmoid(-nsb)))
    pred = jnp.dot(ent, wtref[...], preferred_element_type=jnp.float32)
    pred = pred + bref[...]
    s_ent = jnp.sum((pred - tref[...]) ** 2)
    lane = lax.broadcasted_iota(jnp.int32, (1, 128), 1)
    v = jnp.where(lane == 0, s_skip, jnp.where(lane == 1, s_ent, 0.0))

    @pl.when(i == 0)
    def _():
        oref[...] = v

    @pl.when(i > 0)
    def _():
        oref[...] += v


def _tc_compute(center_all, pos_ctx, neg_ctx, ent_targets, w_t, b2):
    B = center_all.shape[0]
    K = 2 * neg_ctx.shape[0] // B
    R = 512
    nb = B // R
    return pl.pallas_call(
        _tc_body,
        grid=(nb,),
        in_specs=[
            pl.BlockSpec((R, 2 * SEM_DIM), lambda i: (i, 0)),
            pl.BlockSpec((R, SEM_DIM), lambda i: (i, 0)),
            pl.BlockSpec((R * K // 2, 2 * SEM_DIM), lambda i: (i, 0)),
            pl.BlockSpec((R, ENT_DIM), lambda i: (i, 0)),
            pl.BlockSpec((ENT_DIM, ENT_DIM), lambda i: (0, 0)),
            pl.BlockSpec((1, ENT_DIM), lambda i: (0, 0)),
        ],
        out_specs=pl.BlockSpec((1, 128), lambda i: (0, 0)),
        out_shape=jax.ShapeDtypeStruct((1, 128), jnp.float32),
    )(center_all, pos_ctx, neg_ctx, ent_targets, w_t, b2)


def kernel(center_ids, pos_ctx_ids, neg_ctx_ids, ent_targets,
           center_emb, context_emb, W, b):
    B = center_ids.shape[0]
    K = neg_ctx_ids.shape[1]
    cids = center_ids.astype(jnp.int32)
    pids = pos_ctx_ids.astype(jnp.int32)
    nids_flat = neg_ctx_ids.astype(jnp.int32).reshape(B * K)
    center_all, pos_ctx, neg_flat = _sc_gather(
        center_emb, context_emb, cids, pids, nids_flat, K)
    neg_flat = neg_flat.reshape(B * K // 2, 2 * SEM_DIM)
    sums = _tc_compute(center_all, pos_ctx, neg_flat, ent_targets,
                       W.T, b.reshape(1, ENT_DIM))
    skipgram_loss = -sums[0, 0] / B
    ent_loss = sums[0, 1] / (B * ENT_DIM)
    return (skipgram_loss + ent_loss, skipgram_loss, ent_loss)


# speedup vs baseline: 1.2259x; 1.2259x over previous
import functools

import jax
import jax.numpy as jnp
from jax import lax
from jax.experimental import pallas as pl
from jax.experimental.pallas import tpu as pltpu
from jax.experimental.pallas import tpu_sc as plsc

SEM_DIM = 64
ENT_DIM = 64

_NC = 2
_NS = 16
_NW = _NC * _NS

_RC = 64
_GI = 128


def _sc_gather(center_emb, context_emb, center_ids, pos_ids, neg_ids_flat, K):
    B = center_ids.shape[0]
    DC = center_emb.shape[1]
    D = context_emb.shape[1]
    b_w = B // _NW
    n_ch = b_w // _RC
    ng = _RC * K // _GI

    mesh = plsc.VectorSubcoreMesh(
        core_axis_name="c", subcore_axis_name="s",
        num_cores=_NC, num_subcores=_NS)

    @functools.partial(
        pl.kernel,
        out_type=(
            jax.ShapeDtypeStruct((B, DC), jnp.float32),
            jax.ShapeDtypeStruct((B, D), jnp.float32),
            jax.ShapeDtypeStruct((B * K, D), jnp.float32),
        ),
        mesh=mesh,
        scratch_types=[
            pltpu.VMEM((_RC,), jnp.int32),
            pltpu.VMEM((_RC,), jnp.int32),
            pltpu.VMEM((_RC * K,), jnp.int32),
            pltpu.VMEM((_RC, DC), jnp.float32),
            pltpu.VMEM((_RC, D), jnp.float32),
            pltpu.VMEM((_RC * K, D), jnp.float32),
            pltpu.SemaphoreType.DMA,
            pltpu.SemaphoreType.DMA,
        ],
        compiler_params=pltpu.CompilerParams(use_tc_tiling_on_sc=False),
    )
    def k(cemb, xemb, cids, pids, nids, cout, pout, nout,
          cidx, pidx, nidx, crows, prows, nrows, sem_g, sem_w):
        wid = lax.axis_index("s") * _NC + lax.axis_index("c")
        for ch in range(n_ch):
            base = wid * b_w + ch * _RC
            nbase = base * K
            pltpu.sync_copy(cids.at[pl.ds(base, _RC)], cidx)
            pltpu.sync_copy(pids.at[pl.ds(base, _RC)], pidx)
            pltpu.sync_copy(nids.at[pl.ds(nbase, _RC * K)], nidx)
            copies = [
                pltpu.async_copy(cemb.at[cidx], crows, sem_g),
                pltpu.async_copy(xemb.at[pidx], prows, sem_g),
            ]
            for g in range(ng):
                copies.append(pltpu.async_copy(
                    xemb.at[nidx.at[pl.ds(g * _GI, _GI)]],
                    nrows.at[pl.ds(g * _GI, _GI)], sem_g))
            for c in copies:
                c.wait()
            writes = [
                pltpu.async_copy(crows, cout.at[pl.ds(base, _RC)], sem_w),
                pltpu.async_copy(prows, pout.at[pl.ds(base, _RC)], sem_w),
                pltpu.async_copy(nrows, nout.at[pl.ds(nbase, _RC * K)], sem_w),
            ]
            for c in writes:
                c.wait()

    return k(center_emb, context_emb, center_ids, pos_ids, neg_ids_flat)


def _tc_body(cref, pref, nref, tref, wtref, bref, oref):
    i = pl.program_id(0)
    c = cref[...]
    sem = c[:, :SEM_DIM]
    ent = c[:, SEM_DIM:]
    pos = pref[...]
    negf = nref[...]
    r = sem.shape[0]
    kh = negf.shape[0] // r
    semcat = jnp.concatenate([sem, sem], axis=1)
    semrep = jnp.broadcast_to(
        semcat[:, None, :], (r, kh, 2 * SEM_DIM)).reshape(r * kh, 2 * SEM_DIM)
    prodf = negf * semrep
    ri = lax.broadcasted_iota(jnp.int32, (2 * SEM_DIM, 2), 0)
    ci = lax.broadcasted_iota(jnp.int32, (2 * SEM_DIM, 2), 1)
    sel = ((ri < SEM_DIM) == (ci == 0)).astype(jnp.float32)
    ns2 = jnp.dot(prodf, sel, preferred_element_type=jnp.float32)
    ps = jnp.sum(sem * pos, axis=1)

    def log_sigmoid(x):
        return jnp.minimum(x, 0.0) - jnp.log(1.0 + jnp.exp(-jnp.abs(x)))

    s_skip = jnp.sum(log_sigmoid(ps)) + jnp.sum(log_sigmoid(-ns2))
    pred = jnp.dot(ent, wtref[...], preferred_element_type=jnp.float32)
    pred = pred + bref[...]
    s_ent = jnp.sum((pred - tref[...]) ** 2)
    lane = lax.broadcasted_iota(jnp.int32, (1, 128), 1)
    v = jnp.where(lane == 0, s_skip, jnp.where(lane == 1, s_ent, 0.0))

    @pl.when(i == 0)
    def _():
        oref[...] = v

    @pl.when(i > 0)
    def _():
        oref[...] += v


def _tc_compute(center_all, pos_ctx, neg_ctx, ent_targets, w_t, b2):
    B = center_all.shape[0]
    K = 2 * neg_ctx.shape[0] // B
    R = 512
    nb = B // R
    return pl.pallas_call(
        _tc_body,
        grid=(nb,),
        in_specs=[
            pl.BlockSpec((R, 2 * SEM_DIM), lambda i: (i, 0)),
            pl.BlockSpec((R, SEM_DIM), lambda i: (i, 0)),
            pl.BlockSpec((R * K // 2, 2 * SEM_DIM), lambda i: (i, 0)),
            pl.BlockSpec((R, ENT_DIM), lambda i: (i, 0)),
            pl.BlockSpec((ENT_DIM, ENT_DIM), lambda i: (0, 0)),
            pl.BlockSpec((1, ENT_DIM), lambda i: (0, 0)),
        ],
        out_specs=pl.BlockSpec((1, 128), lambda i: (0, 0)),
        out_shape=jax.ShapeDtypeStruct((1, 128), jnp.float32),
    )(center_all, pos_ctx, neg_ctx, ent_targets, w_t, b2)


def kernel(center_ids, pos_ctx_ids, neg_ctx_ids, ent_targets,
           center_emb, context_emb, W, b):
    B = center_ids.shape[0]
    K = neg_ctx_ids.shape[1]
    cids = center_ids.astype(jnp.int32)
    pids = pos_ctx_ids.astype(jnp.int32)
    nids_flat = neg_ctx_ids.astype(jnp.int32).reshape(B * K)
    center_all, pos_ctx, neg_flat = _sc_gather(
        center_emb, context_emb, cids, pids, nids_flat, K)
    neg_flat = neg_flat.reshape(B * K // 2, 2 * SEM_DIM)
    sums = _tc_compute(center_all, pos_ctx, neg_flat, ent_targets,
                       W.T, b.reshape(1, ENT_DIM))
    skipgram_loss = -sums[0, 0] / B
    ent_loss = sums[0, 1] / (B * ENT_DIM)
    return (skipgram_loss + ent_loss, skipgram_loss, ent_loss)


# speedup vs baseline: 1.5386x; 1.2551x over previous
import functools

import jax
import jax.numpy as jnp
from jax import lax
from jax.experimental import pallas as pl
from jax.experimental.pallas import tpu as pltpu
from jax.experimental.pallas import tpu_sc as plsc

SEM_DIM = 64
ENT_DIM = 64

_NC = 2
_NS = 16
_NW = _NC * _NS

_RC = 32
_GI = 128

_FC = 4096


def _fmt_body(xref, oref):
    x = xref[...]
    xt = x.T
    oref[:, :SEM_DIM] = xt
    oref[:, SEM_DIM:] = jnp.zeros_like(xt)


def _ctx_format(ctx_t):
    V = ctx_t.shape[1]
    nb = (V + _FC - 1) // _FC
    return pl.pallas_call(
        _fmt_body,
        grid=(nb,),
        in_specs=[pl.BlockSpec((SEM_DIM, _FC), lambda i: (0, i))],
        out_specs=pl.BlockSpec((_FC, 2 * SEM_DIM), lambda i: (i, 0)),
        out_shape=jax.ShapeDtypeStruct((V, 2 * SEM_DIM), jnp.float32),
    )(ctx_t)


def _sc_gather(center_emb, ctx128, center_ids, pos_ids, neg_ids_flat, K):
    B = center_ids.shape[0]
    DC = center_emb.shape[1]
    D2 = ctx128.shape[1]
    b_w = B // _NW
    n_ch = b_w // _RC
    ng = _RC * K // _GI

    mesh = plsc.VectorSubcoreMesh(
        core_axis_name="c", subcore_axis_name="s",
        num_cores=_NC, num_subcores=_NS)

    @functools.partial(
        pl.kernel,
        out_type=(
            jax.ShapeDtypeStruct((B, DC), jnp.float32),
            jax.ShapeDtypeStruct((B, D2), jnp.float32),
            jax.ShapeDtypeStruct((B * K, D2), jnp.float32),
        ),
        mesh=mesh,
        scratch_types=[
            pltpu.VMEM((_RC,), jnp.int32),
            pltpu.VMEM((_RC,), jnp.int32),
            pltpu.VMEM((_RC * K,), jnp.int32),
            pltpu.VMEM((_RC, DC), jnp.float32),
            pltpu.VMEM((_RC, D2), jnp.float32),
            pltpu.VMEM((_RC * K, D2), jnp.float32),
            pltpu.SemaphoreType.DMA,
            pltpu.SemaphoreType.DMA,
        ],
        compiler_params=pltpu.CompilerParams(use_tc_tiling_on_sc=False),
    )
    def k(cemb, xemb, cids, pids, nids, cout, pout, nout,
          cidx, pidx, nidx, crows, prows, nrows, sem_g, sem_w):
        wid = lax.axis_index("s") * _NC + lax.axis_index("c")
        for ch in range(n_ch):
            base = wid * b_w + ch * _RC
            nbase = base * K
            pltpu.sync_copy(cids.at[pl.ds(base, _RC)], cidx)
            pltpu.sync_copy(pids.at[pl.ds(base, _RC)], pidx)
            pltpu.sync_copy(nids.at[pl.ds(nbase, _RC * K)], nidx)
            copies = [
                pltpu.async_copy(cemb.at[cidx], crows, sem_g),
                pltpu.async_copy(xemb.at[pidx], prows, sem_g),
            ]
            for g in range(ng):
                copies.append(pltpu.async_copy(
                    xemb.at[nidx.at[pl.ds(g * _GI, _GI)]],
                    nrows.at[pl.ds(g * _GI, _GI)], sem_g))
            for c in copies:
                c.wait()
            writes = [
                pltpu.async_copy(crows, cout.at[pl.ds(base, _RC)], sem_w),
                pltpu.async_copy(prows, pout.at[pl.ds(base, _RC)], sem_w),
                pltpu.async_copy(nrows, nout.at[pl.ds(nbase, _RC * K)], sem_w),
            ]
            for c in writes:
                c.wait()

    return k(center_emb, ctx128, center_ids, pos_ids, neg_ids_flat)


def _tc_body(cref, pref, nref, tref, wtref, bref, oref):
    i = pl.program_id(0)
    c = cref[...]
    sem = c[:, :SEM_DIM]
    ent = c[:, SEM_DIM:]
    pos = pref[...]
    negf = nref[...]
    r = sem.shape[0]
    kk = negf.shape[0] // r
    semcat = jnp.concatenate([sem, sem], axis=1)
    ps = jnp.sum(pos * semcat, axis=1)
    semrep = jnp.broadcast_to(
        semcat[:, None, :], (r, kk, 2 * SEM_DIM)).reshape(r * kk, 2 * SEM_DIM)
    ns = jnp.sum(negf * semrep, axis=1)

    def log_sigmoid(x):
        return jnp.minimum(x, 0.0) - jnp.log(1.0 + jnp.exp(-jnp.abs(x)))

    s_skip = jnp.sum(log_sigmoid(ps)) + jnp.sum(log_sigmoid(-ns))
    pred = jnp.dot(ent, wtref[...], preferred_element_type=jnp.float32)
    pred = pred + bref[...]
    s_ent = jnp.sum((pred - tref[...]) ** 2)
    lane = lax.broadcasted_iota(jnp.int32, (1, 128), 1)
    v = jnp.where(lane == 0, s_skip, jnp.where(lane == 1, s_ent, 0.0))

    @pl.when(i == 0)
    def _():
        oref[...] = v

    @pl.when(i > 0)
    def _():
        oref[...] += v


def _tc_compute(center_all, pos128, neg128, ent_targets, w_t, b2):
    B = center_all.shape[0]
    K = neg128.shape[0] // B
    R = 512
    nb = B // R
    return pl.pallas_call(
        _tc_body,
        grid=(nb,),
        in_specs=[
            pl.BlockSpec((R, 2 * SEM_DIM), lambda i: (i, 0)),
            pl.BlockSpec((R, 2 * SEM_DIM), lambda i: (i, 0)),
            pl.BlockSpec((R * K, 2 * SEM_DIM), lambda i: (i, 0)),
            pl.BlockSpec((R, ENT_DIM), lambda i: (i, 0)),
            pl.BlockSpec((ENT_DIM, ENT_DIM), lambda i: (0, 0)),
            pl.BlockSpec((1, ENT_DIM), lambda i: (0, 0)),
        ],
        out_specs=pl.BlockSpec((1, 128), lambda i: (0, 0)),
        out_shape=jax.ShapeDtypeStruct((1, 128), jnp.float32),
    )(center_all, pos128, neg128, ent_targets, w_t, b2)


def kernel(center_ids, pos_ctx_ids, neg_ctx_ids, ent_targets,
           center_emb, context_emb, W, b):
    B = center_ids.shape[0]
    K = neg_ctx_ids.shape[1]
    cids = center_ids.astype(jnp.int32)
    pids = pos_ctx_ids.astype(jnp.int32)
    nids_flat = neg_ctx_ids.astype(jnp.int32).reshape(B * K)
    ctx128 = _ctx_format(context_emb.T)
    center_all, pos128, neg128 = _sc_gather(
        center_emb, ctx128, cids, pids, nids_flat, K)
    sums = _tc_compute(center_all, pos128, neg128, ent_targets,
                       W.T, b.reshape(1, ENT_DIM))
    skipgram_loss = -sums[0, 0] / B
    ent_loss = sums[0, 1] / (B * ENT_DIM)
    return (skipgram_loss + ent_loss, skipgram_loss, ent_loss)


# speedup vs baseline: 1.7384x; 1.1299x over previous
import functools

import jax
import jax.numpy as jnp
from jax import lax
from jax.experimental import pallas as pl
from jax.experimental.pallas import tpu as pltpu
from jax.experimental.pallas import tpu_sc as plsc

SEM_DIM = 64
ENT_DIM = 64

_NC = 2
_NS = 16
_NW = _NC * _NS

_RC = 32
_GI = 128

_FC = 8192


def _fmt_body(xref, oref):
    x = xref[...]
    xt = x.T
    oref[:, :SEM_DIM] = xt
    oref[:, SEM_DIM:] = jnp.zeros_like(xt)


def _ctx_format(ctx_t):
    V = ctx_t.shape[1]
    nb = (V + _FC - 1) // _FC
    return pl.pallas_call(
        _fmt_body,
        grid=(nb,),
        in_specs=[pl.BlockSpec((SEM_DIM, _FC), lambda i: (0, i))],
        out_specs=pl.BlockSpec((_FC, 2 * SEM_DIM), lambda i: (i, 0)),
        out_shape=jax.ShapeDtypeStruct((V, 2 * SEM_DIM), jnp.float32),
    )(ctx_t)


def _sc_gather(center_emb, ctx128, center_ids, pos_ids, neg_ids_flat, K):
    B = center_ids.shape[0]
    DC = center_emb.shape[1]
    D2 = ctx128.shape[1]
    b_w = B // _NW
    n_ch = b_w // _RC
    ng = _RC * K // _GI

    mesh = plsc.VectorSubcoreMesh(
        core_axis_name="c", subcore_axis_name="s",
        num_cores=_NC, num_subcores=_NS)

    @functools.partial(
        pl.kernel,
        out_type=(
            jax.ShapeDtypeStruct((B, DC), jnp.float32),
            jax.ShapeDtypeStruct((B, D2), jnp.float32),
            jax.ShapeDtypeStruct((B * K, D2), jnp.float32),
        ),
        mesh=mesh,
        scratch_types=[
            pltpu.VMEM((_RC,), jnp.int32),
            pltpu.VMEM((_RC,), jnp.int32),
            pltpu.VMEM((_RC * K,), jnp.int32),
            pltpu.VMEM((_RC, DC), jnp.float32),
            pltpu.VMEM((_RC, D2), jnp.float32),
            pltpu.VMEM((_RC * K, D2), jnp.float32),
            pltpu.SemaphoreType.DMA,
            pltpu.SemaphoreType.DMA,
        ],
        compiler_params=pltpu.CompilerParams(use_tc_tiling_on_sc=False),
    )
    def k(cemb, xemb, cids, pids, nids, cout, pout, nout,
          cidx, pidx, nidx, crows, prows, nrows, sem_g, sem_w):
        wid = lax.axis_index("s") * _NC + lax.axis_index("c")
        for ch in range(n_ch):
            base = wid * b_w + ch * _RC
            nbase = base * K
            pltpu.sync_copy(cids.at[pl.ds(base, _RC)], cidx)
            pltpu.sync_copy(pids.at[pl.ds(base, _RC)], pidx)
            pltpu.sync_copy(nids.at[pl.ds(nbase, _RC * K)], nidx)
            copies = [
                pltpu.async_copy(cemb.at[cidx], crows, sem_g),
                pltpu.async_copy(xemb.at[pidx], prows, sem_g),
            ]
            for g in range(ng):
                copies.append(pltpu.async_copy(
                    xemb.at[nidx.at[pl.ds(g * _GI, _GI)]],
                    nrows.at[pl.ds(g * _GI, _GI)], sem_g))
            for c in copies:
                c.wait()
            writes = [
                pltpu.async_copy(crows, cout.at[pl.ds(base, _RC)], sem_w),
                pltpu.async_copy(prows, pout.at[pl.ds(base, _RC)], sem_w),
                pltpu.async_copy(nrows, nout.at[pl.ds(nbase, _RC * K)], sem_w),
            ]
            for c in writes:
                c.wait()

    return k(center_emb, ctx128, center_ids, pos_ids, neg_ids_flat)


def _tc_body(cref, pref, nref, tref, wtref, bref, oref):
    i = pl.program_id(0)
    c = cref[...]
    sem = c[:, :SEM_DIM]
    ent = c[:, SEM_DIM:]
    pos = pref[...]
    negf = nref[...]
    r = sem.shape[0]
    kk = negf.shape[0] // r
    semcat = jnp.concatenate([sem, sem], axis=1)
    ps = jnp.sum(pos * semcat, axis=1)
    semrep = jnp.broadcast_to(
        semcat[:, None, :], (r, kk, 2 * SEM_DIM)).reshape(r * kk, 2 * SEM_DIM)
    ns = jnp.sum(negf * semrep, axis=1)

    def log_sigmoid(x):
        return jnp.minimum(x, 0.0) - jnp.log(1.0 + jnp.exp(-jnp.abs(x)))

    s_skip = jnp.sum(log_sigmoid(ps)) + jnp.sum(log_sigmoid(-ns))
    pred = jnp.dot(ent, wtref[...], preferred_element_type=jnp.float32)
    pred = pred + bref[...]
    s_ent = jnp.sum((pred - tref[...]) ** 2)
    lane = lax.broadcasted_iota(jnp.int32, (1, 128), 1)
    v = jnp.where(lane == 0, s_skip, jnp.where(lane == 1, s_ent, 0.0))

    @pl.when(i == 0)
    def _():
        oref[...] = v

    @pl.when(i > 0)
    def _():
        oref[...] += v


def _tc_compute(center_all, pos128, neg128, ent_targets, w_t, b2):
    B = center_all.shape[0]
    K = neg128.shape[0] // B
    R = 512
    nb = B // R
    return pl.pallas_call(
        _tc_body,
        grid=(nb,),
        in_specs=[
            pl.BlockSpec((R, 2 * SEM_DIM), lambda i: (i, 0)),
            pl.BlockSpec((R, 2 * SEM_DIM), lambda i: (i, 0)),
            pl.BlockSpec((R * K, 2 * SEM_DIM), lambda i: (i, 0)),
            pl.BlockSpec((R, ENT_DIM), lambda i: (i, 0)),
            pl.BlockSpec((ENT_DIM, ENT_DIM), lambda i: (0, 0)),
            pl.BlockSpec((1, ENT_DIM), lambda i: (0, 0)),
        ],
        out_specs=pl.BlockSpec((1, 128), lambda i: (0, 0)),
        out_shape=jax.ShapeDtypeStruct((1, 128), jnp.float32),
    )(center_all, pos128, neg128, ent_targets, w_t, b2)


def kernel(center_ids, pos_ctx_ids, neg_ctx_ids, ent_targets,
           center_emb, context_emb, W, b):
    B = center_ids.shape[0]
    K = neg_ctx_ids.shape[1]
    cids = center_ids.astype(jnp.int32)
    pids = pos_ctx_ids.astype(jnp.int32)
    nids_flat = neg_ctx_ids.astype(jnp.int32).reshape(B * K)
    ctx128 = _ctx_format(context_emb.T)
    center_all, pos128, neg128 = _sc_gather(
        center_emb, ctx128, cids, pids, nids_flat, K)
    sums = _tc_compute(center_all, pos128, neg128, ent_targets,
                       W.T, b.reshape(1, ENT_DIM))
    skipgram_loss = -sums[0, 0] / B
    ent_loss = sums[0, 1] / (B * ENT_DIM)
    return (skipgram_loss + ent_loss, skipgram_loss, ent_loss)


# speedup vs baseline: 1.8019x; 1.0365x over previous
import functools

import jax
import jax.numpy as jnp
from jax import lax
from jax.experimental import pallas as pl
from jax.experimental.pallas import tpu as pltpu
from jax.experimental.pallas import tpu_sc as plsc

SEM_DIM = 64
ENT_DIM = 64

_NC = 2
_NS = 16
_NW = _NC * _NS

_RC = 32
_GI = 128

_FC = 16384


def _fmt_body(xref, oref):
    x = xref[...]
    xt = x.T
    oref[:, :SEM_DIM] = xt
    oref[:, SEM_DIM:] = jnp.zeros_like(xt)


def _ctx_format(ctx_t):
    V = ctx_t.shape[1]
    nb = (V + _FC - 1) // _FC
    return pl.pallas_call(
        _fmt_body,
        grid=(nb,),
        in_specs=[pl.BlockSpec((SEM_DIM, _FC), lambda i: (0, i))],
        out_specs=pl.BlockSpec((_FC, 2 * SEM_DIM), lambda i: (i, 0)),
        out_shape=jax.ShapeDtypeStruct((V, 2 * SEM_DIM), jnp.float32),
    )(ctx_t)


def _sc_gather(center_emb, ctx128, center_ids, pos_ids, neg_ids_flat, K):
    B = center_ids.shape[0]
    DC = center_emb.shape[1]
    D2 = ctx128.shape[1]
    b_w = B // _NW
    n_ch = b_w // _RC
    ng = _RC * K // _GI

    mesh = plsc.VectorSubcoreMesh(
        core_axis_name="c", subcore_axis_name="s",
        num_cores=_NC, num_subcores=_NS)

    @functools.partial(
        pl.kernel,
        out_type=(
            jax.ShapeDtypeStruct((B, DC), jnp.float32),
            jax.ShapeDtypeStruct((B, D2), jnp.float32),
            jax.ShapeDtypeStruct((B * K, D2), jnp.float32),
        ),
        mesh=mesh,
        scratch_types=[
            pltpu.VMEM((_RC,), jnp.int32),
            pltpu.VMEM((_RC,), jnp.int32),
            pltpu.VMEM((_RC * K,), jnp.int32),
            pltpu.VMEM((_RC, DC), jnp.float32),
            pltpu.VMEM((_RC, D2), jnp.float32),
            pltpu.VMEM((_RC * K, D2), jnp.float32),
            pltpu.SemaphoreType.DMA,
            pltpu.SemaphoreType.DMA,
        ],
        compiler_params=pltpu.CompilerParams(use_tc_tiling_on_sc=False),
    )
    def k(cemb, xemb, cids, pids, nids, cout, pout, nout,
          cidx, pidx, nidx, crows, prows, nrows, sem_g, sem_w):
        wid = lax.axis_index("s") * _NC + lax.axis_index("c")
        for ch in range(n_ch):
            base = wid * b_w + ch * _RC
            nbase = base * K
            pltpu.sync_copy(cids.at[pl.ds(base, _RC)], cidx)
            pltpu.sync_copy(pids.at[pl.ds(base, _RC)], pidx)
            pltpu.sync_copy(nids.at[pl.ds(nbase, _RC * K)], nidx)
            copies = [
                pltpu.async_copy(cemb.at[cidx], crows, sem_g),
                pltpu.async_copy(xemb.at[pidx], prows, sem_g),
            ]
            for g in range(ng):
                copies.append(pltpu.async_copy(
                    xemb.at[nidx.at[pl.ds(g * _GI, _GI)]],
                    nrows.at[pl.ds(g * _GI, _GI)], sem_g))
            for c in copies:
                c.wait()
            writes = [
                pltpu.async_copy(crows, cout.at[pl.ds(base, _RC)], sem_w),
                pltpu.async_copy(prows, pout.at[pl.ds(base, _RC)], sem_w),
                pltpu.async_copy(nrows, nout.at[pl.ds(nbase, _RC * K)], sem_w),
            ]
            for c in writes:
                c.wait()

    return k(center_emb, ctx128, center_ids, pos_ids, neg_ids_flat)


def _tc_body(cref, pref, nref, tref, wtref, bref, oref):
    i = pl.program_id(0)
    c = cref[...]
    sem = c[:, :SEM_DIM]
    ent = c[:, SEM_DIM:]
    pos = pref[...]
    negf = nref[...]
    r = sem.shape[0]
    kk = negf.shape[0] // r
    semcat = jnp.concatenate([sem, sem], axis=1)
    ps = jnp.sum(pos * semcat, axis=1)
    semrep = jnp.broadcast_to(
        semcat[:, None, :], (r, kk, 2 * SEM_DIM)).reshape(r * kk, 2 * SEM_DIM)
    ns = jnp.sum(negf * semrep, axis=1)

    def log_sigmoid(x):
        return jnp.minimum(x, 0.0) - jnp.log(1.0 + jnp.exp(-jnp.abs(x)))

    s_skip = jnp.sum(log_sigmoid(ps)) + jnp.sum(log_sigmoid(-ns))
    pred = jnp.dot(ent, wtref[...], preferred_element_type=jnp.float32)
    pred = pred + bref[...]
    s_ent = jnp.sum((pred - tref[...]) ** 2)
    lane = lax.broadcasted_iota(jnp.int32, (1, 128), 1)
    v = jnp.where(lane == 0, s_skip, jnp.where(lane == 1, s_ent, 0.0))

    @pl.when(i == 0)
    def _():
        oref[...] = v

    @pl.when(i > 0)
    def _():
        oref[...] += v


def _tc_compute(center_all, pos128, neg128, ent_targets, w_t, b2):
    B = center_all.shape[0]
    K = neg128.shape[0] // B
    R = 512
    nb = B // R
    return pl.pallas_call(
        _tc_body,
        grid=(nb,),
        in_specs=[
            pl.BlockSpec((R, 2 * SEM_DIM), lambda i: (i, 0)),
            pl.BlockSpec((R, 2 * SEM_DIM), lambda i: (i, 0)),
            pl.BlockSpec((R * K, 2 * SEM_DIM), lambda i: (i, 0)),
            pl.BlockSpec((R, ENT_DIM), lambda i: (i, 0)),
            pl.BlockSpec((ENT_DIM, ENT_DIM), lambda i: (0, 0)),
            pl.BlockSpec((1, ENT_DIM), lambda i: (0, 0)),
        ],
        out_specs=pl.BlockSpec((1, 128), lambda i: (0, 0)),
        out_shape=jax.ShapeDtypeStruct((1, 128), jnp.float32),
    )(center_all, pos128, neg128, ent_targets, w_t, b2)


def kernel(center_ids, pos_ctx_ids, neg_ctx_ids, ent_targets,
           center_emb, context_emb, W, b):
    B = center_ids.shape[0]
    K = neg_ctx_ids.shape[1]
    cids = center_ids.astype(jnp.int32)
    pids = pos_ctx_ids.astype(jnp.int32)
    nids_flat = neg_ctx_ids.astype(jnp.int32).reshape(B * K)
    ctx128 = _ctx_format(context_emb.T)
    center_all, pos128, neg128 = _sc_gather(
        center_emb, ctx128, cids, pids, nids_flat, K)
    sums = _tc_compute(center_all, pos128, neg128, ent_targets,
                       W.T, b.reshape(1, ENT_DIM))
    skipgram_loss = -sums[0, 0] / B
    ent_loss = sums[0, 1] / (B * ENT_DIM)
    return (skipgram_loss + ent_loss, skipgram_loss, ent_loss)


# speedup vs baseline: 1.8937x; 1.0509x over previous
import functools

import jax
import jax.numpy as jnp
from jax import lax
from jax.experimental import pallas as pl
from jax.experimental.pallas import tpu as pltpu
from jax.experimental.pallas import tpu_sc as plsc

SEM_DIM = 64
ENT_DIM = 64

_NC = 2
_NS = 16
_NW = _NC * _NS

_RC = 16
_GI = 80

_FC = 16384


def _fmt_body(xref, oref):
    x = xref[...]
    xt = x.T
    oref[:, :SEM_DIM] = xt
    oref[:, SEM_DIM:] = jnp.zeros_like(xt)


def _ctx_format(ctx_t):
    V = ctx_t.shape[1]
    nb = (V + _FC - 1) // _FC
    return pl.pallas_call(
        _fmt_body,
        grid=(nb,),
        in_specs=[pl.BlockSpec((SEM_DIM, _FC), lambda i: (0, i))],
        out_specs=pl.BlockSpec((_FC, 2 * SEM_DIM), lambda i: (i, 0)),
        out_shape=jax.ShapeDtypeStruct((V, 2 * SEM_DIM), jnp.float32),
    )(ctx_t)


def _sc_gather(center_emb, ctx128, center_ids, pos_ids, neg_ids_flat, K):
    B = center_ids.shape[0]
    DC = center_emb.shape[1]
    D2 = ctx128.shape[1]
    b_w = B // _NW
    n_ch = b_w // _RC
    ng = _RC * K // _GI

    mesh = plsc.VectorSubcoreMesh(
        core_axis_name="c", subcore_axis_name="s",
        num_cores=_NC, num_subcores=_NS)

    @functools.partial(
        pl.kernel,
        out_type=(
            jax.ShapeDtypeStruct((B, DC), jnp.float32),
            jax.ShapeDtypeStruct((B, D2), jnp.float32),
            jax.ShapeDtypeStruct((B * K, D2), jnp.float32),
        ),
        mesh=mesh,
        scratch_types=[
            pltpu.VMEM((b_w,), jnp.int32),
            pltpu.VMEM((b_w,), jnp.int32),
            pltpu.VMEM((b_w * K,), jnp.int32),
            pltpu.VMEM((2, _RC, DC), jnp.float32),
            pltpu.VMEM((2, _RC, D2), jnp.float32),
            pltpu.VMEM((2, _RC * K, D2), jnp.float32),
            pltpu.SemaphoreType.DMA,
            pltpu.SemaphoreType.DMA,
            pltpu.SemaphoreType.DMA,
            pltpu.SemaphoreType.DMA,
        ],
        compiler_params=pltpu.CompilerParams(use_tc_tiling_on_sc=False),
    )
    def k(cemb, xemb, cids, pids, nids, cout, pout, nout,
          cidx, pidx, nidx, crows, prows, nrows, sg0, sg1, sw0, sw1):
        wid = lax.axis_index("s") * _NC + lax.axis_index("c")
        base0 = wid * b_w
        pltpu.sync_copy(cids.at[pl.ds(base0, b_w)], cidx)
        pltpu.sync_copy(pids.at[pl.ds(base0, b_w)], pidx)
        pltpu.sync_copy(nids.at[pl.ds(base0 * K, b_w * K)], nidx)
        sems_g = (sg0, sg1)
        sems_w = (sw0, sw1)
        pending = {0: [], 1: []}
        for ch in range(n_ch):
            s = ch % 2
            base = base0 + ch * _RC
            nbase = base * K
            for c in pending[s]:
                c.wait()
            pending[s] = []
            copies = [
                pltpu.async_copy(cemb.at[cidx.at[pl.ds(ch * _RC, _RC)]],
                                 crows.at[s], sems_g[s]),
                pltpu.async_copy(xemb.at[pidx.at[pl.ds(ch * _RC, _RC)]],
                                 prows.at[s], sems_g[s]),
            ]
            for g in range(ng):
                copies.append(pltpu.async_copy(
                    xemb.at[nidx.at[pl.ds(ch * _RC * K + g * _GI, _GI)]],
                    nrows.at[s].at[pl.ds(g * _GI, _GI)], sems_g[s]))
            for c in copies:
                c.wait()
            pending[s] = [
                pltpu.async_copy(crows.at[s], cout.at[pl.ds(base, _RC)],
                                 sems_w[s]),
                pltpu.async_copy(prows.at[s], pout.at[pl.ds(base, _RC)],
                                 sems_w[s]),
                pltpu.async_copy(nrows.at[s], nout.at[pl.ds(nbase, _RC * K)],
                                 sems_w[s]),
            ]
        for s in (0, 1):
            for c in pending[s]:
                c.wait()

    return k(center_emb, ctx128, center_ids, pos_ids, neg_ids_flat)


def _tc_body(cref, pref, nref, tref, wtref, bref, oref):
    i = pl.program_id(0)
    c = cref[...]
    sem = c[:, :SEM_DIM]
    ent = c[:, SEM_DIM:]
    pos = pref[...]
    negf = nref[...]
    r = sem.shape[0]
    kk = negf.shape[0] // r
    semcat = jnp.concatenate([sem, sem], axis=1)
    ps = jnp.sum(pos * semcat, axis=1)
    semrep = jnp.broadcast_to(
        semcat[:, None, :], (r, kk, 2 * SEM_DIM)).reshape(r * kk, 2 * SEM_DIM)
    ns = jnp.sum(negf * semrep, axis=1)

    def log_sigmoid(x):
        return jnp.minimum(x, 0.0) - jnp.log(1.0 + jnp.exp(-jnp.abs(x)))

    s_skip = jnp.sum(log_sigmoid(ps)) + jnp.sum(log_sigmoid(-ns))
    pred = jnp.dot(ent, wtref[...], preferred_element_type=jnp.float32)
    pred = pred + bref[...]
    s_ent = jnp.sum((pred - tref[...]) ** 2)
    lane = lax.broadcasted_iota(jnp.int32, (1, 128), 1)
    v = jnp.where(lane == 0, s_skip, jnp.where(lane == 1, s_ent, 0.0))

    @pl.when(i == 0)
    def _():
        oref[...] = v

    @pl.when(i > 0)
    def _():
        oref[...] += v


def _tc_compute(center_all, pos128, neg128, ent_targets, w_t, b2):
    B = center_all.shape[0]
    K = neg128.shape[0] // B
    R = 512
    nb = B // R
    return pl.pallas_call(
        _tc_body,
        grid=(nb,),
        in_specs=[
            pl.BlockSpec((R, 2 * SEM_DIM), lambda i: (i, 0)),
            pl.BlockSpec((R, 2 * SEM_DIM), lambda i: (i, 0)),
            pl.BlockSpec((R * K, 2 * SEM_DIM), lambda i: (i, 0)),
            pl.BlockSpec((R, ENT_DIM), lambda i: (i, 0)),
            pl.BlockSpec((ENT_DIM, ENT_DIM), lambda i: (0, 0)),
            pl.BlockSpec((1, ENT_DIM), lambda i: (0, 0)),
        ],
        out_specs=pl.BlockSpec((1, 128), lambda i: (0, 0)),
        out_shape=jax.ShapeDtypeStruct((1, 128), jnp.float32),
    )(center_all, pos128, neg128, ent_targets, w_t, b2)


def kernel(center_ids, pos_ctx_ids, neg_ctx_ids, ent_targets,
           center_emb, context_emb, W, b):
    B = center_ids.shape[0]
    K = neg_ctx_ids.shape[1]
    cids = center_ids.astype(jnp.int32)
    pids = pos_ctx_ids.astype(jnp.int32)
    nids_flat = neg_ctx_ids.astype(jnp.int32).reshape(B * K)
    ctx128 = _ctx_format(context_emb.T)
    center_all, pos128, neg128 = _sc_gather(
        center_emb, ctx128, cids, pids, nids_flat, K)
    sums = _tc_compute(center_all, pos128, neg128, ent_targets,
                       W.T, b.reshape(1, ENT_DIM))
    skipgram_loss = -sums[0, 0] / B
    ent_loss = sums[0, 1] / (B * ENT_DIM)
    return (skipgram_loss + ent_loss, skipgram_loss, ent_loss)


# speedup vs baseline: 1.9169x; 1.0123x over previous
import functools

import jax
import jax.numpy as jnp
from jax import lax
from jax.experimental import pallas as pl
from jax.experimental.pallas import tpu as pltpu
from jax.experimental.pallas import tpu_sc as plsc

SEM_DIM = 64
ENT_DIM = 64

_NC = 2
_NS = 16
_NW = _NC * _NS

_RC = 16
_GI = 80

_FC = 16384


def _fmt_body(xref, oref):
    x = xref[...]
    xt = x.T
    oref[:, :SEM_DIM] = xt
    oref[:, SEM_DIM:] = jnp.zeros_like(xt)


def _ctx_format(ctx_t):
    V = ctx_t.shape[1]
    nb = (V + _FC - 1) // _FC
    return pl.pallas_call(
        _fmt_body,
        grid=(nb,),
        in_specs=[pl.BlockSpec((SEM_DIM, _FC), lambda i: (0, i))],
        out_specs=pl.BlockSpec((_FC, 2 * SEM_DIM), lambda i: (i, 0)),
        out_shape=jax.ShapeDtypeStruct((V, 2 * SEM_DIM), jnp.float32),
    )(ctx_t)


def _sc_gather(center_emb, ctx128, center_ids, pos_ids, neg_ids_flat, K):
    B = center_ids.shape[0]
    DC = center_emb.shape[1]
    D2 = ctx128.shape[1]
    b_w = B // _NW
    n_ch = b_w // _RC
    ng = _RC * K // _GI

    mesh = plsc.VectorSubcoreMesh(
        core_axis_name="c", subcore_axis_name="s",
        num_cores=_NC, num_subcores=_NS)

    @functools.partial(
        pl.kernel,
        out_type=(
            jax.ShapeDtypeStruct((B, DC), jnp.float32),
            jax.ShapeDtypeStruct((B, D2), jnp.float32),
            jax.ShapeDtypeStruct((B * K, D2), jnp.float32),
        ),
        mesh=mesh,
        scratch_types=[
            pltpu.VMEM((b_w,), jnp.int32),
            pltpu.VMEM((b_w,), jnp.int32),
            pltpu.VMEM((b_w * K,), jnp.int32),
            pltpu.VMEM((2, _RC, DC), jnp.float32),
            pltpu.VMEM((2, _RC, D2), jnp.float32),
            pltpu.VMEM((2, _RC * K, D2), jnp.float32),
            pltpu.SemaphoreType.DMA,
            pltpu.SemaphoreType.DMA,
            pltpu.SemaphoreType.DMA,
            pltpu.SemaphoreType.DMA,
        ],
        compiler_params=pltpu.CompilerParams(use_tc_tiling_on_sc=False),
    )
    def k(cemb, xemb, cids, pids, nids, cout, pout, nout,
          cidx, pidx, nidx, crows, prows, nrows, sg0, sg1, sw0, sw1):
        wid = lax.axis_index("s") * _NC + lax.axis_index("c")
        base0 = wid * b_w
        pltpu.sync_copy(cids.at[pl.ds(base0, b_w)], cidx)
        pltpu.sync_copy(pids.at[pl.ds(base0, b_w)], pidx)
        pltpu.sync_copy(nids.at[pl.ds(base0 * K, b_w * K)], nidx)
        sems_g = (sg0, sg1)
        sems_w = (sw0, sw1)
        pending = {0: [], 1: []}
        for ch in range(n_ch):
            s = ch % 2
            base = base0 + ch * _RC
            nbase = base * K
            for c in pending[s]:
                c.wait()
            pending[s] = []
            copies = [
                pltpu.async_copy(cemb.at[cidx.at[pl.ds(ch * _RC, _RC)]],
                                 crows.at[s], sems_g[s]),
                pltpu.async_copy(xemb.at[pidx.at[pl.ds(ch * _RC, _RC)]],
                                 prows.at[s], sems_g[s]),
            ]
            for g in range(ng):
                copies.append(pltpu.async_copy(
                    xemb.at[nidx.at[pl.ds(ch * _RC * K + g * _GI, _GI)]],
                    nrows.at[s].at[pl.ds(g * _GI, _GI)], sems_g[s]))
            for c in copies:
                c.wait()
            pending[s] = [
                pltpu.async_copy(crows.at[s], cout.at[pl.ds(base, _RC)],
                                 sems_w[s]),
                pltpu.async_copy(prows.at[s], pout.at[pl.ds(base, _RC)],
                                 sems_w[s]),
                pltpu.async_copy(nrows.at[s], nout.at[pl.ds(nbase, _RC * K)],
                                 sems_w[s]),
            ]
        for s in (0, 1):
            for c in pending[s]:
                c.wait()

    return k(center_emb, ctx128, center_ids, pos_ids, neg_ids_flat)


def _tc_body(cref, pref, nref, tref, wtref, bref, oref):
    i = pl.program_id(0)
    c = cref[...]
    sem = c[:, :SEM_DIM]
    ent = c[:, SEM_DIM:]
    pos = pref[...]
    negf = nref[...]
    r = sem.shape[0]
    kk = negf.shape[0] // r
    semcat = jnp.concatenate([sem, sem], axis=1)
    ps = jnp.sum(pos * semcat, axis=1)
    semrep = jnp.broadcast_to(
        semcat[:, None, :], (r, kk, 2 * SEM_DIM)).reshape(r * kk, 2 * SEM_DIM)
    ns = jnp.sum(negf * semrep, axis=1)

    def log_sigmoid(x):
        return jnp.minimum(x, 0.0) - jnp.log(1.0 + jnp.exp(-jnp.abs(x)))

    s_skip = jnp.sum(log_sigmoid(ps)) + jnp.sum(log_sigmoid(-ns))
    pred = jnp.dot(ent, wtref[...], preferred_element_type=jnp.float32)
    pred = pred + bref[...]
    s_ent = jnp.sum((pred - tref[...]) ** 2)
    lane = lax.broadcasted_iota(jnp.int32, (1, 128), 1)
    v = jnp.where(lane == 0, s_skip, jnp.where(lane == 1, s_ent, 0.0))

    @pl.when(i == 0)
    def _():
        oref[...] = v

    @pl.when(i > 0)
    def _():
        oref[...] += v


def _tc_compute(center_all, pos128, neg128, ent_targets, w_t, b2):
    B = center_all.shape[0]
    K = neg128.shape[0] // B
    R = 512
    nb = B // R
    return pl.pallas_call(
        _tc_body,
        grid=(nb,),
        in_specs=[
            pl.BlockSpec((R, 2 * SEM_DIM), lambda i: (i, 0)),
            pl.BlockSpec((R, 2 * SEM_DIM), lambda i: (i, 0)),
            pl.BlockSpec((R * K, 2 * SEM_DIM), lambda i: (i, 0)),
            pl.BlockSpec((R, ENT_DIM), lambda i: (i, 0)),
            pl.BlockSpec((ENT_DIM, ENT_DIM), lambda i: (0, 0)),
            pl.BlockSpec((1, ENT_DIM), lambda i: (0, 0)),
        ],
        out_specs=pl.BlockSpec((1, 128), lambda i: (0, 0)),
        out_shape=jax.ShapeDtypeStruct((1, 128), jnp.float32),
    )(center_all, pos128, neg128, ent_targets, w_t, b2)


def kernel(center_ids, pos_ctx_ids, neg_ctx_ids, ent_targets,
           center_emb, context_emb, W, b):
    B = center_ids.shape[0]
    K = neg_ctx_ids.shape[1]
    cids = center_ids.astype(jnp.int32)
    pids = pos_ctx_ids.astype(jnp.int32)
    nids_flat = neg_ctx_ids.astype(jnp.int32).reshape(B * K)
    ctx128 = _ctx_format(context_emb.T)
    h = B // 2
    w_t = W.T
    b2 = b.reshape(1, ENT_DIM)
    sums = None
    for s in range(2):
        center_all, pos128, neg128 = _sc_gather(
            center_emb, ctx128, cids[s * h:(s + 1) * h],
            pids[s * h:(s + 1) * h],
            nids_flat[s * h * K:(s + 1) * h * K], K)
        part = _tc_compute(center_all, pos128, neg128,
                           ent_targets[s * h:(s + 1) * h], w_t, b2)
        sums = part if sums is None else sums + part
    skipgram_loss = -sums[0, 0] / B
    ent_loss = sums[0, 1] / (B * ENT_DIM)
    return (skipgram_loss + ent_loss, skipgram_loss, ent_loss)


# speedup vs baseline: 1.9324x; 1.0080x over previous
import functools

import jax
import jax.numpy as jnp
from jax import lax
from jax.experimental import pallas as pl
from jax.experimental.pallas import tpu as pltpu
from jax.experimental.pallas import tpu_sc as plsc

SEM_DIM = 64
ENT_DIM = 64

_NC = 2
_NS = 16
_NW = _NC * _NS

_RC = 16
_GI = 80

_FC = 16384


def _fmt_body(xref, oref):
    x = xref[...]
    xt = x.T
    oref[:, :SEM_DIM] = xt
    oref[:, SEM_DIM:] = jnp.zeros_like(xt)


def _ctx_format(ctx_t):
    V = ctx_t.shape[1]
    nb = (V + _FC - 1) // _FC
    return pl.pallas_call(
        _fmt_body,
        grid=(nb,),
        in_specs=[pl.BlockSpec((SEM_DIM, _FC), lambda i: (0, i))],
        out_specs=pl.BlockSpec((_FC, 2 * SEM_DIM), lambda i: (i, 0)),
        out_shape=jax.ShapeDtypeStruct((V, 2 * SEM_DIM), jnp.float32),
    )(ctx_t)


def _sc_gather(center_emb, ctx128, center_ids, pos_ids, neg_ids_flat, K):
    B = center_ids.shape[0]
    DC = center_emb.shape[1]
    D2 = ctx128.shape[1]
    b_w = B // _NW
    n_ch = b_w // _RC
    ng = _RC * K // _GI

    mesh = plsc.VectorSubcoreMesh(
        core_axis_name="c", subcore_axis_name="s",
        num_cores=_NC, num_subcores=_NS)

    @functools.partial(
        pl.kernel,
        out_type=(
            jax.ShapeDtypeStruct((B, DC), jnp.float32),
            jax.ShapeDtypeStruct((B, D2), jnp.float32),
            jax.ShapeDtypeStruct((B * K, D2), jnp.float32),
        ),
        mesh=mesh,
        scratch_types=[
            pltpu.VMEM((b_w,), jnp.int32),
            pltpu.VMEM((b_w,), jnp.int32),
            pltpu.VMEM((b_w * K,), jnp.int32),
            pltpu.VMEM((2, _RC, DC), jnp.float32),
            pltpu.VMEM((2, _RC, D2), jnp.float32),
            pltpu.VMEM((2, _RC * K, D2), jnp.float32),
            pltpu.SemaphoreType.DMA,
            pltpu.SemaphoreType.DMA,
            pltpu.SemaphoreType.DMA,
            pltpu.SemaphoreType.DMA,
        ],
        compiler_params=pltpu.CompilerParams(use_tc_tiling_on_sc=False),
    )
    def k(cemb, xemb, cids, pids, nids, cout, pout, nout,
          cidx, pidx, nidx, crows, prows, nrows, sg0, sg1, sw0, sw1):
        wid = lax.axis_index("s") * _NC + lax.axis_index("c")
        base0 = wid * b_w
        pltpu.sync_copy(cids.at[pl.ds(base0, b_w)], cidx)
        pltpu.sync_copy(pids.at[pl.ds(base0, b_w)], pidx)
        pltpu.sync_copy(nids.at[pl.ds(base0 * K, b_w * K)], nidx)
        sems_g = (sg0, sg1)
        sems_w = (sw0, sw1)
        pending = {0: [], 1: []}
        for ch in range(n_ch):
            s = ch % 2
            base = base0 + ch * _RC
            nbase = base * K
            for c in pending[s]:
                c.wait()
            pending[s] = []
            copies = [
                pltpu.async_copy(cemb.at[cidx.at[pl.ds(ch * _RC, _RC)]],
                                 crows.at[s], sems_g[s]),
                pltpu.async_copy(xemb.at[pidx.at[pl.ds(ch * _RC, _RC)]],
                                 prows.at[s], sems_g[s]),
            ]
            for g in range(ng):
                copies.append(pltpu.async_copy(
                    xemb.at[nidx.at[pl.ds(ch * _RC * K + g * _GI, _GI)]],
                    nrows.at[s].at[pl.ds(g * _GI, _GI)], sems_g[s]))
            for c in copies:
                c.wait()
            pending[s] = [
                pltpu.async_copy(crows.at[s], cout.at[pl.ds(base, _RC)],
                                 sems_w[s]),
                pltpu.async_copy(prows.at[s], pout.at[pl.ds(base, _RC)],
                                 sems_w[s]),
                pltpu.async_copy(nrows.at[s], nout.at[pl.ds(nbase, _RC * K)],
                                 sems_w[s]),
            ]
        for s in (0, 1):
            for c in pending[s]:
                c.wait()

    return k(center_emb, ctx128, center_ids, pos_ids, neg_ids_flat)


def _tc_body(cref, pref, nref, tref, wtref, bref, oref):
    i = pl.program_id(0)
    c = cref[...]
    sem = c[:, :SEM_DIM]
    ent = c[:, SEM_DIM:]
    pos = pref[...]
    negf = nref[...]
    r = sem.shape[0]
    kk = negf.shape[0] // r
    semcat = jnp.concatenate([sem, sem], axis=1)
    ps = jnp.sum(pos * semcat, axis=1)
    semrep = jnp.broadcast_to(
        semcat[:, None, :], (r, kk, 2 * SEM_DIM)).reshape(r * kk, 2 * SEM_DIM)
    ns = jnp.sum(negf * semrep, axis=1)

    def log_sigmoid(x):
        return jnp.minimum(x, 0.0) - jnp.log(1.0 + jnp.exp(-jnp.abs(x)))

    s_skip = jnp.sum(log_sigmoid(ps)) + jnp.sum(log_sigmoid(-ns))
    pred = jnp.dot(ent, wtref[...], preferred_element_type=jnp.float32)
    pred = pred + bref[...]
    s_ent = jnp.sum((pred - tref[...]) ** 2)
    lane = lax.broadcasted_iota(jnp.int32, (1, 128), 1)
    v = jnp.where(lane == 0, s_skip, jnp.where(lane == 1, s_ent, 0.0))

    @pl.when(i == 0)
    def _():
        oref[...] = v

    @pl.when(i > 0)
    def _():
        oref[...] += v


def _tc_compute(center_all, pos128, neg128, ent_targets, w_t, b2, blk_ofs):
    B = center_all.shape[0]
    K = neg128.shape[0] // B
    R = 512
    nb = B // R
    return pl.pallas_call(
        _tc_body,
        grid=(nb,),
        in_specs=[
            pl.BlockSpec((R, 2 * SEM_DIM), lambda i: (i, 0)),
            pl.BlockSpec((R, 2 * SEM_DIM), lambda i: (i, 0)),
            pl.BlockSpec((R * K, 2 * SEM_DIM), lambda i: (i, 0)),
            pl.BlockSpec((R, ENT_DIM), lambda i: (i + blk_ofs, 0)),
            pl.BlockSpec((ENT_DIM, ENT_DIM), lambda i: (0, 0)),
            pl.BlockSpec((1, ENT_DIM), lambda i: (0, 0)),
        ],
        out_specs=pl.BlockSpec((1, 128), lambda i: (0, 0)),
        out_shape=jax.ShapeDtypeStruct((1, 128), jnp.float32),
    )(center_all, pos128, neg128, ent_targets, w_t, b2)


def kernel(center_ids, pos_ctx_ids, neg_ctx_ids, ent_targets,
           center_emb, context_emb, W, b):
    B = center_ids.shape[0]
    K = neg_ctx_ids.shape[1]
    cids = center_ids.astype(jnp.int32)
    pids = pos_ctx_ids.astype(jnp.int32)
    nids_flat = neg_ctx_ids.astype(jnp.int32).reshape(B * K)
    ctx128 = _ctx_format(context_emb.T)
    n_sl = 4
    h = B // n_sl
    w_t = W.T
    b2 = b.reshape(1, ENT_DIM)
    sums = None
    for s in range(n_sl):
        center_all, pos128, neg128 = _sc_gather(
            center_emb, ctx128, cids[s * h:(s + 1) * h],
            pids[s * h:(s + 1) * h],
            nids_flat[s * h * K:(s + 1) * h * K], K)
        part = _tc_compute(center_all, pos128, neg128,
                           ent_targets, w_t, b2, s * (h // 512))
        sums = part if sums is None else sums + part
    skipgram_loss = -sums[0, 0] / B
    ent_loss = sums[0, 1] / (B * ENT_DIM)
    return (skipgram_loss + ent_loss, skipgram_loss, ent_loss)
